# baseline (speedup 1.0000x reference)
import functools
import jax
import jax.numpy as jnp
from jax.experimental import pallas as pl

KNOW = 128
EXER = 20000
STU = 100000

_BLK = 2048


def _mm_body(h_ref, w1_ref, w2_ref, A1_ref, A2_ref,
             z1_ref, z2_ref, s1_ref, s2_ref):
    h = h_ref[...]
    z1 = jnp.dot(h, w1_ref[...], preferred_element_type=jnp.float32)
    z2 = jnp.dot(h, w2_ref[...], preferred_element_type=jnp.float32)
    z1_ref[...] = z1
    z2_ref[...] = z2
    s1_ref[...] = jnp.dot(z1, A1_ref[...], preferred_element_type=jnp.float32)
    s2_ref[...] = jnp.dot(z2, A2_ref[...], preferred_element_type=jnp.float32)


def _dense_stage(h, W1, W2, a1, a2):
    n = h.shape[0]
    npad = ((n + _BLK - 1) // _BLK) * _BLK
    hp = jnp.pad(h, ((0, npad - n), (0, 0)))
    A1 = jnp.concatenate([a1[:KNOW], a1[KNOW:], jnp.zeros((KNOW, KNOW - 2), jnp.float32)], axis=1)
    A2 = jnp.concatenate([a2[:KNOW], a2[KNOW:], jnp.zeros((KNOW, KNOW - 2), jnp.float32)], axis=1)
    grid = npad // _BLK
    full = pl.BlockSpec((KNOW, KNOW), lambda i: (0, 0))
    blk = pl.BlockSpec((_BLK, KNOW), lambda i: (i, 0))
    z1, z2, s1, s2 = pl.pallas_call(
        _mm_body,
        grid=(grid,),
        in_specs=[blk, full, full, full, full],
        out_specs=[blk, blk, blk, blk],
        out_shape=[jax.ShapeDtypeStruct((npad, KNOW), jnp.float32)] * 4,
    )(hp, W1, W2, A1, A2)
    return (z1[:n], z2[:n], s1[:n, 0], s1[:n, 1], s2[:n, 0], s2[:n, 1])


def _edge_softmax_agg(z, p, q, src, dst, n):
    e = jax.nn.leaky_relu(p[src] + q[dst], 0.01)
    emax = jax.ops.segment_max(e, dst, num_segments=n)
    emax = jnp.where(jnp.isfinite(emax), emax, 0.0)
    ex = jnp.exp(e - emax[dst])
    denom = jax.ops.segment_sum(ex, dst, num_segments=n)
    alpha = ex / (denom[dst] + 1e-16)
    return jax.ops.segment_sum(alpha[:, None] * z[src], dst, num_segments=n)


def kernel(kn_emb, exer_emb, all_stu_emb, W_kfe, a_kfe, W_efk, a_efk, W_sfe,
           a_sfe, W_efs, a_efs, k_attn_w3, k_attn_b3, e_attn_w1, e_attn_b1,
           e_attn_w2, e_attn_b2, src_kfe, dst_kfe, src_efk, dst_efk, src_sfe,
           dst_sfe, src_efs, dst_efs):
    n_ek = EXER + KNOW
    n_es = EXER + STU
    h_ek = jnp.concatenate([exer_emb, kn_emb], axis=0)
    h_es = jnp.concatenate([exer_emb, all_stu_emb], axis=0)

    z_kfe, z_efk, p_kfe, q_kfe, p_efk, q_efk = _dense_stage(h_ek, W_kfe, W_efk, a_kfe, a_efk)
    z_sfe, z_efs, p_sfe, q_sfe, p_efs, q_efs = _dense_stage(h_es, W_sfe, W_efs, a_sfe, a_efs)

    k_from_e = _edge_softmax_agg(z_kfe, p_kfe, q_kfe, src_kfe, dst_kfe, n_ek)
    e_from_k = _edge_softmax_agg(z_efk, p_efk, q_efk, src_efk, dst_efk, n_ek)
    u_from_e = _edge_softmax_agg(z_sfe, p_sfe, q_sfe, src_sfe, dst_sfe, n_es)
    e_from_u = _edge_softmax_agg(z_efs, p_efs, q_efs, src_efs, dst_efs, n_es)

    A = kn_emb
    D = k_from_e[EXER:]
    score3 = jax.nn.softmax(jnp.concatenate([A, D], axis=1) @ k_attn_w3 + k_attn_b3, axis=1)
    kn_out = A + score3[:, 0:1] * D

    A = exer_emb
    B = e_from_k[:EXER]
    C = e_from_u[:EXER]
    s1 = jnp.concatenate([A, B], axis=1) @ e_attn_w1 + e_attn_b1
    s2 = jnp.concatenate([A, C], axis=1) @ e_attn_w2 + e_attn_b2
    score = jax.nn.softmax(jnp.concatenate([s1, s2], axis=1), axis=1)
    exer_out = exer_emb + score[:, 0:1] * B + score[:, 1:2] * C

    stu_out = all_stu_emb + u_from_e[EXER:]
    return (kn_out, exer_out, stu_out)


# speedup vs baseline: 1.7097x; 1.7097x over previous
import functools
import jax
import jax.numpy as jnp
from jax.experimental import pallas as pl

KNOW = 128
EXER = 20000
STU = 100000

_BLK = 2048


def _mm_body(h_ref, w1_ref, w2_ref, A1_ref, A2_ref,
             z1_ref, z2_ref, s1_ref, s2_ref):
    h = h_ref[...]
    z1 = jnp.dot(h, w1_ref[...], preferred_element_type=jnp.float32)
    z2 = jnp.dot(h, w2_ref[...], preferred_element_type=jnp.float32)
    z1_ref[...] = z1
    z2_ref[...] = z2
    s1_ref[...] = jnp.dot(z1, A1_ref[...], preferred_element_type=jnp.float32)
    s2_ref[...] = jnp.dot(z2, A2_ref[...], preferred_element_type=jnp.float32)


def _dense_stage(h, W1, W2, a1, a2):
    n = h.shape[0]
    npad = ((n + _BLK - 1) // _BLK) * _BLK
    hp = jnp.pad(h, ((0, npad - n), (0, 0)))
    A1 = jnp.concatenate([a1[:KNOW], a1[KNOW:], jnp.zeros((KNOW, KNOW - 2), jnp.float32)], axis=1)
    A2 = jnp.concatenate([a2[:KNOW], a2[KNOW:], jnp.zeros((KNOW, KNOW - 2), jnp.float32)], axis=1)
    grid = npad // _BLK
    full = pl.BlockSpec((KNOW, KNOW), lambda i: (0, 0))
    blk = pl.BlockSpec((_BLK, KNOW), lambda i: (i, 0))
    z1, z2, s1, s2 = pl.pallas_call(
        _mm_body,
        grid=(grid,),
        in_specs=[blk, full, full, full, full],
        out_specs=[blk, blk, blk, blk],
        out_shape=[jax.ShapeDtypeStruct((npad, KNOW), jnp.float32)] * 4,
    )(hp, W1, W2, A1, A2)
    return (z1[:n], z2[:n], s1[:n, 0], s1[:n, 1], s2[:n, 0], s2[:n, 1])


def _edge_softmax_agg(z, p, q, src, dst, n):
    e = jax.nn.leaky_relu(p[src] + q[dst], 0.01)
    ex = jnp.exp(e)
    denom = jax.ops.segment_sum(ex, dst, num_segments=n)
    u = jax.ops.segment_sum(ex[:, None] * z[src], dst, num_segments=n)
    return u / (denom[:, None] + 1e-16)


def kernel(kn_emb, exer_emb, all_stu_emb, W_kfe, a_kfe, W_efk, a_efk, W_sfe,
           a_sfe, W_efs, a_efs, k_attn_w3, k_attn_b3, e_attn_w1, e_attn_b1,
           e_attn_w2, e_attn_b2, src_kfe, dst_kfe, src_efk, dst_efk, src_sfe,
           dst_sfe, src_efs, dst_efs):
    n_ek = EXER + KNOW
    n_es = EXER + STU
    h_ek = jnp.concatenate([exer_emb, kn_emb], axis=0)
    h_es = jnp.concatenate([exer_emb, all_stu_emb], axis=0)

    z_kfe, z_efk, p_kfe, q_kfe, p_efk, q_efk = _dense_stage(h_ek, W_kfe, W_efk, a_kfe, a_efk)
    z_sfe, z_efs, p_sfe, q_sfe, p_efs, q_efs = _dense_stage(h_es, W_sfe, W_efs, a_sfe, a_efs)

    k_from_e = _edge_softmax_agg(z_kfe, p_kfe, q_kfe, src_kfe, dst_kfe, n_ek)
    e_from_k = _edge_softmax_agg(z_efk, p_efk, q_efk, src_efk, dst_efk, n_ek)
    u_from_e = _edge_softmax_agg(z_sfe, p_sfe, q_sfe, src_sfe, dst_sfe, n_es)
    e_from_u = _edge_softmax_agg(z_efs, p_efs, q_efs, src_efs, dst_efs, n_es)

    A = kn_emb
    D = k_from_e[EXER:]
    score3 = jax.nn.softmax(jnp.concatenate([A, D], axis=1) @ k_attn_w3 + k_attn_b3, axis=1)
    kn_out = A + score3[:, 0:1] * D

    A = exer_emb
    B = e_from_k[:EXER]
    C = e_from_u[:EXER]
    s1 = jnp.concatenate([A, B], axis=1) @ e_attn_w1 + e_attn_b1
    s2 = jnp.concatenate([A, C], axis=1) @ e_attn_w2 + e_attn_b2
    score = jax.nn.softmax(jnp.concatenate([s1, s2], axis=1), axis=1)
    exer_out = exer_emb + score[:, 0:1] * B + score[:, 1:2] * C

    stu_out = all_stu_emb + u_from_e[EXER:]
    return (kn_out, exer_out, stu_out)


# speedup vs baseline: 3.0921x; 1.8085x over previous
import functools
import jax
import jax.numpy as jnp
from jax.experimental import pallas as pl

KNOW = 128
EXER = 20000
STU = 100000

_BLK = 2048


def _mm_body(h_ref, w1_ref, w2_ref, A1_ref, A2_ref,
             z1_ref, z2_ref, s1_ref, s2_ref):
    h = h_ref[...]
    z1 = jnp.dot(h, w1_ref[...], preferred_element_type=jnp.float32)
    z2 = jnp.dot(h, w2_ref[...], preferred_element_type=jnp.float32)
    z1_ref[...] = z1
    z2_ref[...] = z2
    s1_ref[...] = jnp.dot(z1, A1_ref[...], preferred_element_type=jnp.float32)
    s2_ref[...] = jnp.dot(z2, A2_ref[...], preferred_element_type=jnp.float32)


def _dense_stage(h, W1, W2, a1, a2):
    n = h.shape[0]
    npad = ((n + _BLK - 1) // _BLK) * _BLK
    hp = jnp.pad(h, ((0, npad - n), (0, 0)))
    A1 = jnp.concatenate([a1[:KNOW], a1[KNOW:], jnp.zeros((KNOW, KNOW - 2), jnp.float32)], axis=1)
    A2 = jnp.concatenate([a2[:KNOW], a2[KNOW:], jnp.zeros((KNOW, KNOW - 2), jnp.float32)], axis=1)
    grid = npad // _BLK
    full = pl.BlockSpec((KNOW, KNOW), lambda i: (0, 0))
    blk = pl.BlockSpec((_BLK, KNOW), lambda i: (i, 0))
    z1, z2, s1, s2 = pl.pallas_call(
        _mm_body,
        grid=(grid,),
        in_specs=[blk, full, full, full, full],
        out_specs=[blk, blk, blk, blk],
        out_shape=[jax.ShapeDtypeStruct((npad, KNOW), jnp.float32)] * 4,
    )(hp, W1, W2, A1, A2)
    return (z1[:n], z2[:n], s1[:n, 0], s1[:n, 1], s2[:n, 0], s2[:n, 1])


def _edge_softmax_agg(z, p, q, src, dst, n):
    zc = jnp.concatenate([z, jnp.ones((z.shape[0], 1), z.dtype), p[:, None]], axis=1)
    g = zc[src]
    e = jax.nn.leaky_relu(g[:, 129] + q[dst], 0.01)
    ex = jnp.exp(e)
    uc = jax.ops.segment_sum(ex[:, None] * g[:, :129], dst, num_segments=n)
    return uc[:, :128] / (uc[:, 128:129] + 1e-16)


def kernel(kn_emb, exer_emb, all_stu_emb, W_kfe, a_kfe, W_efk, a_efk, W_sfe,
           a_sfe, W_efs, a_efs, k_attn_w3, k_attn_b3, e_attn_w1, e_attn_b1,
           e_attn_w2, e_attn_b2, src_kfe, dst_kfe, src_efk, dst_efk, src_sfe,
           dst_sfe, src_efs, dst_efs):
    n_ek = EXER + KNOW
    n_es = EXER + STU
    h_ek = jnp.concatenate([exer_emb, kn_emb], axis=0)
    h_es = jnp.concatenate([exer_emb, all_stu_emb], axis=0)

    z_kfe, z_efk, p_kfe, q_kfe, p_efk, q_efk = _dense_stage(h_ek, W_kfe, W_efk, a_kfe, a_efk)
    z_sfe, z_efs, p_sfe, q_sfe, p_efs, q_efs = _dense_stage(h_es, W_sfe, W_efs, a_sfe, a_efs)

    k_from_e = _edge_softmax_agg(z_kfe, p_kfe, q_kfe, src_kfe, dst_kfe, n_ek)
    e_from_k = _edge_softmax_agg(z_efk, p_efk, q_efk, src_efk, dst_efk, n_ek)
    u_from_e = _edge_softmax_agg(z_sfe, p_sfe, q_sfe, src_sfe, dst_sfe, n_es)
    e_from_u = _edge_softmax_agg(z_efs, p_efs, q_efs, src_efs, dst_efs, n_es)

    A = kn_emb
    D = k_from_e[EXER:]
    score3 = jax.nn.softmax(jnp.concatenate([A, D], axis=1) @ k_attn_w3 + k_attn_b3, axis=1)
    kn_out = A + score3[:, 0:1] * D

    A = exer_emb
    B = e_from_k[:EXER]
    C = e_from_u[:EXER]
    s1 = jnp.concatenate([A, B], axis=1) @ e_attn_w1 + e_attn_b1
    s2 = jnp.concatenate([A, C], axis=1) @ e_attn_w2 + e_attn_b2
    score = jax.nn.softmax(jnp.concatenate([s1, s2], axis=1), axis=1)
    exer_out = exer_emb + score[:, 0:1] * B + score[:, 1:2] * C

    stu_out = all_stu_emb + u_from_e[EXER:]
    return (kn_out, exer_out, stu_out)
